# baseline (speedup 1.0000x reference)
import functools

import jax
import jax.numpy as jnp
from jax import lax
from jax.experimental import pallas as pl
from jax.experimental.pallas import tpu as pltpu
from jax.experimental.pallas import tpu_sc as plsc

VOCAB = 1000000
D = 64
HIDDEN = 256
OUT = 50
B = 4096
L = 50

NC = 2
NS = 16
NW = NC * NS
BAGS_PER_W = B // NW
LANE = 16
DCH = D // LANE
BLK = 16
NBLK = BAGS_PER_W // BLK
IW = 64
BB = BLK * IW
CBAG = 1
CI = CBAG * IW
NCH = BLK // CBAG
PGRP = 8192
PNG = (VOCAB + PGRP - 1) // PGRP
PSUB = PGRP // 128
PROWS = PNG * PSUB * D


def _pack_body(embt_ref, out_ref):
    blk = embt_ref[...]
    ident = jnp.eye(D, dtype=jnp.float32)
    blkt = lax.dot_general(
        blk, ident, (((0,), (0,)), ((), ())),
        preferred_element_type=jnp.float32,
    )
    for s in range(PSUB):
        out_ref[0, s, :, 0:D] = blkt[128 * s:128 * s + D]
        out_ref[0, s, :, D:2 * D] = blkt[128 * s + D:128 * (s + 1)]


def _pack(embT):
    return pl.pallas_call(
        _pack_body,
        grid=(PNG,),
        in_specs=[pl.BlockSpec((D, PGRP), lambda k: (0, k))],
        out_specs=pl.BlockSpec((1, PSUB, D, 128), lambda k: (k, 0, 0, 0)),
        out_shape=jax.ShapeDtypeStruct((PNG, PSUB, D, 128), jnp.float32),
    )(embT)


def _sc_bag_sum_body(idx_hbm, table_hbm, out_hbm,
                     idx_v, tidx_v, rv_v, rows_v, wb_v,
                     gsem0, gsem1, isem0, isem1, wsem0, wsem1):
    wid = lax.axis_index("s") * NC + lax.axis_index("c")
    bbase = wid * BAGS_PER_W
    ibase = wid * (BAGS_PER_W * IW)

    gsems = (gsem0, gsem1)
    isems = (isem0, isem1)
    wsems = (wsem0, wsem1)

    def idx_dma(blk, buf):
        return pltpu.make_async_copy(
            idx_hbm.at[pl.ds(ibase + blk * BB, BB)],
            idx_v.at[pl.ds(buf * BB, BB)],
            isems[buf],
        )

    def tconv(buf):
        for t in range(BLK):
            for u in range(IW // LANE):
                o = buf * BB + t * IW + u * LANE
                v = idx_v[pl.ds(o, LANE)]
                tidx_v[pl.ds(o, LANE)] = (
                    lax.shift_right_logical(v, 7) * D + (v & (D - 1))
                )
                rv_v[pl.ds(o, LANE)] = (lax.shift_right_logical(v, 6) & 1) * D

    def gather(buf, cbuf, ch):
        base = buf * BB + pl.multiple_of(ch * CI, IW)
        return pltpu.make_async_copy(
            table_hbm.at[tidx_v.at[pl.ds(base, CI)]],
            rows_v.at[cbuf],
            gsems[cbuf],
        )

    def wb_copy(blk, buf):
        return pltpu.make_async_copy(
            wb_v.at[buf],
            out_hbm.at[pl.ds(bbase + blk * BLK, BLK), :],
            wsems[buf],
        )

    def accumulate(buf, cbuf, ch):
        if True:
            gb = 0
            zero = jnp.zeros((LANE,), jnp.float32)
            gl = ch * CBAG + gb
            base = buf * BB + pl.multiple_of(gl * IW, LANE)
            rbase = 0

            def grp(t, acc):
                rch = rv_v[pl.ds(base + t * LANE, LANE)]
                for u in range(LANE):
                    j = t * LANE + u
                    r = rch[u]
                    acc = tuple(
                        acc[i]
                        + rows_v[cbuf, rbase + j,
                                 pl.ds(pl.multiple_of(r + i * LANE, LANE), LANE)]
                        for i in range(DCH)
                    )
                return acc

            acc = lax.fori_loop(0, L // LANE, grp, (zero,) * DCH)
            rch = rv_v[pl.ds(base + (L // LANE) * LANE, LANE)]
            for u in range(L % LANE):
                j = (L // LANE) * LANE + u
                r = rch[u]
                acc = tuple(
                    acc[i]
                    + rows_v[cbuf, rbase + j,
                             pl.ds(pl.multiple_of(r + i * LANE, LANE), LANE)]
                    for i in range(DCH)
                )
            for i in range(DCH):
                wb_v[buf, gl, pl.ds(i * LANE, LANE)] = acc[i]

    idx_dma(0, 0).start()
    idx_dma(0, 0).wait()
    tconv(0)
    idx_dma(1, 1).start()
    gather(0, 0, 0).start()

    def sb_body(sb, carry):
        for ib in range(2):
            b = sb * 2 + ib

            @pl.when(b >= 2)
            def _():
                wb_copy(b - 2, ib).wait()

            @pl.when(b + 1 < NBLK)
            def _():
                idx_dma(b + 1, 1 - ib).wait()
                tconv(1 - ib)

            @pl.when(b + 2 < NBLK)
            def _():
                idx_dma(b + 2, ib).start()

            def chpair(cp, _c):
                for cc in range(2):
                    ch = cp * 2 + cc
                    cbuf = cc
                    gather(ib, cbuf, ch).wait()
                    if cc == 0:
                        gather(ib, 1, ch + 1).start()
                    else:
                        @pl.when(cp < NCH // 2 - 1)
                        def _():
                            gather(ib, 0, ch + 1).start()

                        @pl.when((cp == NCH // 2 - 1) & (b + 1 < NBLK))
                        def _():
                            gather(1 - ib, 0, 0).start()
                    accumulate(ib, cbuf, ch)
                return _c

            lax.fori_loop(0, NCH // 2, chpair, 0)
            wb_copy(b, ib).start()
        return carry

    lax.fori_loop(0, NBLK // 2, sb_body, 0)

    wb_copy(NBLK - 2, 0).wait()
    wb_copy(NBLK - 1, 1).wait()


@functools.cache
def _sc_bag_sum():
    return pl.kernel(
        _sc_bag_sum_body,
        out_type=jax.ShapeDtypeStruct((B, D), jnp.float32),
        mesh=plsc.VectorSubcoreMesh(
            core_axis_name="c", subcore_axis_name="s", num_cores=NC, num_subcores=NS
        ),
        scratch_types=[
            pltpu.VMEM((2 * BB,), jnp.int32),
            pltpu.VMEM((2 * BB,), jnp.int32),
            pltpu.VMEM((2 * BB,), jnp.int32),
            pltpu.VMEM((2, CI, 128), jnp.float32),
            pltpu.VMEM((2, BLK, D), jnp.float32),
            pltpu.SemaphoreType.DMA,
            pltpu.SemaphoreType.DMA,
            pltpu.SemaphoreType.DMA,
            pltpu.SemaphoreType.DMA,
            pltpu.SemaphoreType.DMA,
            pltpu.SemaphoreType.DMA,
        ],
    )


def _ffnn_body(vec_ref, w1_ref, b1_ref, w2_ref, b2_ref, out_ref):
    x = vec_ref[...] * (1.0 / L)
    h = jnp.dot(x, w1_ref[...], preferred_element_type=jnp.float32) + b1_ref[...]
    h = jnp.maximum(h, 0.0)
    logits = jnp.dot(h, w2_ref[...], preferred_element_type=jnp.float32) + b2_ref[...]
    m = jnp.max(logits, axis=1, keepdims=True)
    shifted = logits - m
    lse = jnp.log(jnp.sum(jnp.exp(shifted), axis=1, keepdims=True))
    out_ref[...] = shifted - lse


def _ffnn(vec, W1, b1, W2, b2):
    blk = 512
    grid = (B // blk,)
    return pl.pallas_call(
        _ffnn_body,
        grid=grid,
        in_specs=[
            pl.BlockSpec((blk, D), lambda i: (i, 0)),
            pl.BlockSpec((D, HIDDEN), lambda i: (0, 0)),
            pl.BlockSpec((1, HIDDEN), lambda i: (0, 0)),
            pl.BlockSpec((HIDDEN, OUT), lambda i: (0, 0)),
            pl.BlockSpec((1, OUT), lambda i: (0, 0)),
        ],
        out_specs=pl.BlockSpec((blk, OUT), lambda i: (i, 0)),
        out_shape=jax.ShapeDtypeStruct((B, OUT), jnp.float32),
    )(vec, W1, b1, W2, b2)


def kernel(indices, embeddings, W1, b1, W2, b2):
    packed = _pack(embeddings.T).reshape(PROWS, 128)
    ipad = jnp.pad(indices, ((0, 0), (0, IW - L)), mode='wrap').reshape(-1)
    vec_sum = _sc_bag_sum()(ipad, packed)
    return _ffnn(vec_sum, W1, b1.reshape(1, -1), W2, b2.reshape(1, -1))


# speedup vs baseline: 1.1159x; 1.1159x over previous
import functools

import jax
import jax.numpy as jnp
from jax import lax
from jax.experimental import pallas as pl
from jax.experimental.pallas import tpu as pltpu
from jax.experimental.pallas import tpu_sc as plsc

VOCAB = 1000000
D = 64
HIDDEN = 256
OUT = 50
B = 4096
L = 50

NC = 2
NS = 16
NW = NC * NS
BAGS_PER_W = B // NW
LANE = 16
DCH = D // LANE
BLK = 16
NBLK = BAGS_PER_W // BLK
IW = 64
BB = BLK * IW
PGRP = 16384
PNG = (VOCAB + PGRP - 1) // PGRP
PSUB = PGRP // 128
PROWS = PNG * PSUB * D


def _pack_body(embt_ref, out_ref):
    blk = embt_ref[...]
    ident = jnp.eye(D, dtype=jnp.float32)
    blkt = lax.dot_general(
        blk, ident, (((0,), (0,)), ((), ())),
        preferred_element_type=jnp.float32,
    )
    for s in range(PSUB):
        out_ref[0, s, :, 0:D] = blkt[128 * s:128 * s + D]
        out_ref[0, s, :, D:2 * D] = blkt[128 * s + D:128 * (s + 1)]


def _pack(embT):
    return pl.pallas_call(
        _pack_body,
        grid=(PNG,),
        in_specs=[pl.BlockSpec((D, PGRP), lambda k: (0, k))],
        out_specs=pl.BlockSpec((1, PSUB, D, 128), lambda k: (k, 0, 0, 0)),
        out_shape=jax.ShapeDtypeStruct((PNG, PSUB, D, 128), jnp.float32),
    )(embT)


def _sc_bag_sum_body(idx_hbm, table_hbm, out_hbm,
                     idx_v, tidx_v, rv_v, rows_v, wb_v,
                     gsem0, gsem1, isem0, isem1, wsem0, wsem1):
    wid = lax.axis_index("s") * NC + lax.axis_index("c")
    bbase = wid * BAGS_PER_W
    ibase = wid * (BAGS_PER_W * IW)

    gsems = (gsem0, gsem1)
    isems = (isem0, isem1)
    wsems = (wsem0, wsem1)

    def idx_dma(blk, buf):
        return pltpu.make_async_copy(
            idx_hbm.at[pl.ds(ibase + blk * BB, BB)],
            idx_v.at[pl.ds(buf * BB, BB)],
            isems[buf],
        )

    def tconv(buf):
        for t in range(BLK):
            for u in range(IW // LANE):
                o = buf * BB + t * IW + u * LANE
                v = idx_v[pl.ds(o, LANE)]
                tidx_v[pl.ds(o, LANE)] = (
                    lax.shift_right_logical(v, 7) * D + (v & (D - 1))
                )
                rv_v[pl.ds(o, LANE)] = (lax.shift_right_logical(v, 6) & 1) * D

    def gather(buf, rbuf, row):
        base = buf * BB + pl.multiple_of(row * IW, IW)
        return pltpu.make_async_copy(
            table_hbm.at[tidx_v.at[pl.ds(base, L)]],
            rows_v.at[rbuf],
            gsems[rbuf],
        )

    def wb_copy(blk, buf):
        return pltpu.make_async_copy(
            wb_v.at[buf],
            out_hbm.at[pl.ds(bbase + blk * BLK, BLK), :],
            wsems[buf],
        )

    def accumulate(buf, rbuf, g_local):
        zero = jnp.zeros((LANE,), jnp.float32)
        base = buf * BB + pl.multiple_of(g_local * IW, LANE)

        def grp(t, acc):
            rch = rv_v[pl.ds(base + t * LANE, LANE)]
            for u in range(LANE):
                j = t * LANE + u
                r = rch[u]
                acc = tuple(
                    acc[i]
                    + rows_v[rbuf, j,
                             pl.ds(pl.multiple_of(r + i * LANE, LANE), LANE)]
                    for i in range(DCH)
                )
            return acc

        acc = lax.fori_loop(0, L // LANE, grp, (zero,) * DCH)
        rch = rv_v[pl.ds(base + (L // LANE) * LANE, LANE)]
        for u in range(L % LANE):
            j = (L // LANE) * LANE + u
            r = rch[u]
            acc = tuple(
                acc[i]
                + rows_v[rbuf, j,
                         pl.ds(pl.multiple_of(r + i * LANE, LANE), LANE)]
                for i in range(DCH)
            )
        for i in range(DCH):
            wb_v[buf, g_local, pl.ds(i * LANE, LANE)] = acc[i]

    idx_dma(0, 0).start()
    idx_dma(0, 0).wait()
    tconv(0)
    idx_dma(1, 1).start()
    gather(0, 0, 0).start()

    def sb_body(sb, carry):
        for ib in range(2):
            b = sb * 2 + ib

            @pl.when(b >= 2)
            def _():
                wb_copy(b - 2, ib).wait()

            @pl.when(b + 1 < NBLK)
            def _():
                idx_dma(b + 1, 1 - ib).wait()
                tconv(1 - ib)

            @pl.when(b + 2 < NBLK)
            def _():
                idx_dma(b + 2, ib).start()

            def pair(bp, _2):
                for b2 in range(2):
                    g_local = bp * 2 + b2
                    rbuf = b2
                    gather(ib, rbuf, g_local).wait()
                    if b2 == 0:
                        gather(ib, 1, g_local + 1).start()
                    else:
                        @pl.when(bp < BLK // 2 - 1)
                        def _():
                            gather(ib, 0, g_local + 1).start()

                        @pl.when((bp == BLK // 2 - 1) & (b + 1 < NBLK))
                        def _():
                            gather(1 - ib, 0, 0).start()
                    accumulate(ib, rbuf, g_local)
                return _2

            lax.fori_loop(0, BLK // 2, pair, 0)
            wb_copy(b, ib).start()
        return carry

    lax.fori_loop(0, NBLK // 2, sb_body, 0)

    wb_copy(NBLK - 2, 0).wait()
    wb_copy(NBLK - 1, 1).wait()


@functools.cache
def _sc_bag_sum():
    return pl.kernel(
        _sc_bag_sum_body,
        out_type=jax.ShapeDtypeStruct((B, D), jnp.float32),
        mesh=plsc.VectorSubcoreMesh(
            core_axis_name="c", subcore_axis_name="s", num_cores=NC, num_subcores=NS
        ),
        scratch_types=[
            pltpu.VMEM((2 * BB,), jnp.int32),
            pltpu.VMEM((2 * BB,), jnp.int32),
            pltpu.VMEM((2 * BB,), jnp.int32),
            pltpu.VMEM((2, L, 2 * D), jnp.float32),
            pltpu.VMEM((2, BLK, D), jnp.float32),
            pltpu.SemaphoreType.DMA,
            pltpu.SemaphoreType.DMA,
            pltpu.SemaphoreType.DMA,
            pltpu.SemaphoreType.DMA,
            pltpu.SemaphoreType.DMA,
            pltpu.SemaphoreType.DMA,
        ],
    )


def _ffnn_body(vec_ref, w1_ref, b1_ref, w2_ref, b2_ref, out_ref):
    x = vec_ref[...] * (1.0 / L)
    h = jnp.dot(x, w1_ref[...], preferred_element_type=jnp.float32) + b1_ref[...]
    h = jnp.maximum(h, 0.0)
    logits = jnp.dot(h, w2_ref[...], preferred_element_type=jnp.float32) + b2_ref[...]
    m = jnp.max(logits, axis=1, keepdims=True)
    shifted = logits - m
    lse = jnp.log(jnp.sum(jnp.exp(shifted), axis=1, keepdims=True))
    out_ref[...] = shifted - lse


def _ffnn(vec, W1, b1, W2, b2):
    blk = 512
    grid = (B // blk,)
    return pl.pallas_call(
        _ffnn_body,
        grid=grid,
        in_specs=[
            pl.BlockSpec((blk, D), lambda i: (i, 0)),
            pl.BlockSpec((D, HIDDEN), lambda i: (0, 0)),
            pl.BlockSpec((1, HIDDEN), lambda i: (0, 0)),
            pl.BlockSpec((HIDDEN, OUT), lambda i: (0, 0)),
            pl.BlockSpec((1, OUT), lambda i: (0, 0)),
        ],
        out_specs=pl.BlockSpec((blk, OUT), lambda i: (i, 0)),
        out_shape=jax.ShapeDtypeStruct((B, OUT), jnp.float32),
    )(vec, W1, b1, W2, b2)


def kernel(indices, embeddings, W1, b1, W2, b2):
    packed = _pack(embeddings.T).reshape(PROWS, 128)
    ipad = jnp.pad(indices, ((0, 0), (0, IW - L))).reshape(-1)
    vec_sum = _sc_bag_sum()(ipad, packed)
    return _ffnn(vec_sum, W1, b1.reshape(1, -1), W2, b2.reshape(1, -1))


# speedup vs baseline: 1.1605x; 1.0399x over previous
import functools

import jax
import jax.numpy as jnp
from jax import lax
from jax.experimental import pallas as pl
from jax.experimental.pallas import tpu as pltpu
from jax.experimental.pallas import tpu_sc as plsc

VOCAB = 1000000
D = 64
HIDDEN = 256
OUT = 50
B = 4096
L = 50

NC = 2
NS = 16
NW = NC * NS
BAGS_PER_W = B // NW
LANE = 16
DCH = D // LANE
BLK = 16
NBLK = BAGS_PER_W // BLK
IW = 64
BB = BLK * IW
PGRP = 32768
PNG = (VOCAB + PGRP - 1) // PGRP
PSUB = PGRP // 128
PROWS = PNG * PSUB * D


def _pack_body(embt_ref, out_ref):
    blk = embt_ref[...]
    ident = jnp.eye(D, dtype=jnp.float32)
    blkt = lax.dot_general(
        blk, ident, (((0,), (0,)), ((), ())),
        preferred_element_type=jnp.float32,
    )
    for s in range(PSUB):
        out_ref[0, s, :, 0:D] = blkt[128 * s:128 * s + D]
        out_ref[0, s, :, D:2 * D] = blkt[128 * s + D:128 * (s + 1)]


def _pack(embT):
    return pl.pallas_call(
        _pack_body,
        grid=(PNG,),
        in_specs=[pl.BlockSpec((D, PGRP), lambda k: (0, k))],
        out_specs=pl.BlockSpec((1, PSUB, D, 128), lambda k: (k, 0, 0, 0)),
        out_shape=jax.ShapeDtypeStruct((PNG, PSUB, D, 128), jnp.float32),
    )(embT)


def _sc_bag_sum_body(idx_hbm, table_hbm, out_hbm,
                     idx_v, tidx_v, rv_v, rows_v, wb_v,
                     gsem0, gsem1, isem0, isem1, wsem0, wsem1):
    wid = lax.axis_index("s") * NC + lax.axis_index("c")
    bbase = wid * BAGS_PER_W
    ibase = wid * (BAGS_PER_W * IW)

    gsems = (gsem0, gsem1)
    isems = (isem0, isem1)
    wsems = (wsem0, wsem1)

    def idx_dma(blk, buf):
        return pltpu.make_async_copy(
            idx_hbm.at[pl.ds(ibase + blk * BB, BB)],
            idx_v.at[pl.ds(buf * BB, BB)],
            isems[buf],
        )

    def tconv(buf):
        for t in range(BLK):
            for u in range(IW // LANE):
                o = buf * BB + t * IW + u * LANE
                v = idx_v[pl.ds(o, LANE)]
                tidx_v[pl.ds(o, LANE)] = (
                    lax.shift_right_logical(v, 7) * D + (v & (D - 1))
                )
                rv_v[pl.ds(o, LANE)] = (lax.shift_right_logical(v, 6) & 1) * D

    def gather(buf, rbuf, row):
        base = buf * BB + pl.multiple_of(row * IW, IW)
        return pltpu.make_async_copy(
            table_hbm.at[tidx_v.at[pl.ds(base, L)]],
            rows_v.at[rbuf],
            gsems[rbuf],
        )

    def wb_copy(blk, buf):
        return pltpu.make_async_copy(
            wb_v.at[buf],
            out_hbm.at[pl.ds(bbase + blk * BLK, BLK), :],
            wsems[buf],
        )

    def accumulate(buf, rbuf, g_local):
        zero = jnp.zeros((LANE,), jnp.float32)
        base = buf * BB + pl.multiple_of(g_local * IW, LANE)

        def grp(t, acc):
            rch = rv_v[pl.ds(base + t * LANE, LANE)]
            for u in range(LANE):
                j = t * LANE + u
                r = rch[u]
                acc = tuple(
                    acc[i]
                    + rows_v[rbuf, j,
                             pl.ds(pl.multiple_of(r + i * LANE, LANE), LANE)]
                    for i in range(DCH)
                )
            return acc

        acc = lax.fori_loop(0, L // LANE, grp, (zero,) * DCH)
        rch = rv_v[pl.ds(base + (L // LANE) * LANE, LANE)]
        for u in range(L % LANE):
            j = (L // LANE) * LANE + u
            r = rch[u]
            acc = tuple(
                acc[i]
                + rows_v[rbuf, j,
                         pl.ds(pl.multiple_of(r + i * LANE, LANE), LANE)]
                for i in range(DCH)
            )
        for i in range(DCH):
            wb_v[buf, g_local, pl.ds(i * LANE, LANE)] = acc[i]

    idx_dma(0, 0).start()
    idx_dma(0, 0).wait()
    tconv(0)
    idx_dma(1, 1).start()
    gather(0, 0, 0).start()

    def sb_body(sb, carry):
        for ib in range(2):
            b = sb * 2 + ib

            @pl.when(b >= 2)
            def _():
                wb_copy(b - 2, ib).wait()

            @pl.when(b + 1 < NBLK)
            def _():
                idx_dma(b + 1, 1 - ib).wait()
                tconv(1 - ib)

            @pl.when(b + 2 < NBLK)
            def _():
                idx_dma(b + 2, ib).start()

            def pair(bp, _2):
                for b2 in range(2):
                    g_local = bp * 2 + b2
                    rbuf = b2
                    gather(ib, rbuf, g_local).wait()
                    if b2 == 0:
                        gather(ib, 1, g_local + 1).start()
                    else:
                        @pl.when(bp < BLK // 2 - 1)
                        def _():
                            gather(ib, 0, g_local + 1).start()

                        @pl.when((bp == BLK // 2 - 1) & (b + 1 < NBLK))
                        def _():
                            gather(1 - ib, 0, 0).start()
                    accumulate(ib, rbuf, g_local)
                return _2

            lax.fori_loop(0, BLK // 2, pair, 0)
            wb_copy(b, ib).start()
        return carry

    lax.fori_loop(0, NBLK // 2, sb_body, 0)

    wb_copy(NBLK - 2, 0).wait()
    wb_copy(NBLK - 1, 1).wait()


@functools.cache
def _sc_bag_sum():
    return pl.kernel(
        _sc_bag_sum_body,
        out_type=jax.ShapeDtypeStruct((B, D), jnp.float32),
        mesh=plsc.VectorSubcoreMesh(
            core_axis_name="c", subcore_axis_name="s", num_cores=NC, num_subcores=NS
        ),
        scratch_types=[
            pltpu.VMEM((2 * BB,), jnp.int32),
            pltpu.VMEM((2 * BB,), jnp.int32),
            pltpu.VMEM((2 * BB,), jnp.int32),
            pltpu.VMEM((2, L, 2 * D), jnp.float32),
            pltpu.VMEM((2, BLK, D), jnp.float32),
            pltpu.SemaphoreType.DMA,
            pltpu.SemaphoreType.DMA,
            pltpu.SemaphoreType.DMA,
            pltpu.SemaphoreType.DMA,
            pltpu.SemaphoreType.DMA,
            pltpu.SemaphoreType.DMA,
        ],
    )


def _ffnn_body(vec_ref, w1_ref, b1_ref, w2_ref, b2_ref, out_ref):
    x = vec_ref[...] * (1.0 / L)
    h = jnp.dot(x, w1_ref[...], preferred_element_type=jnp.float32) + b1_ref[...]
    h = jnp.maximum(h, 0.0)
    logits = jnp.dot(h, w2_ref[...], preferred_element_type=jnp.float32) + b2_ref[...]
    m = jnp.max(logits, axis=1, keepdims=True)
    shifted = logits - m
    lse = jnp.log(jnp.sum(jnp.exp(shifted), axis=1, keepdims=True))
    out_ref[...] = shifted - lse


def _ffnn(vec, W1, b1, W2, b2):
    blk = 512
    grid = (B // blk,)
    return pl.pallas_call(
        _ffnn_body,
        grid=grid,
        in_specs=[
            pl.BlockSpec((blk, D), lambda i: (i, 0)),
            pl.BlockSpec((D, HIDDEN), lambda i: (0, 0)),
            pl.BlockSpec((1, HIDDEN), lambda i: (0, 0)),
            pl.BlockSpec((HIDDEN, OUT), lambda i: (0, 0)),
            pl.BlockSpec((1, OUT), lambda i: (0, 0)),
        ],
        out_specs=pl.BlockSpec((blk, OUT), lambda i: (i, 0)),
        out_shape=jax.ShapeDtypeStruct((B, OUT), jnp.float32),
    )(vec, W1, b1, W2, b2)


def kernel(indices, embeddings, W1, b1, W2, b2):
    packed = _pack(embeddings.T).reshape(PROWS, 128)
    ipad = jnp.pad(indices, ((0, 0), (0, IW - L))).reshape(-1)
    vec_sum = _sc_bag_sum()(ipad, packed)
    return _ffnn(vec_sum, W1, b1.reshape(1, -1), W2, b2.reshape(1, -1))
